# baseline (speedup 1.0000x reference)
import functools

import jax
import jax.numpy as jnp
from jax import lax
from jax.experimental import pallas as pl
from jax.experimental.pallas import tpu as pltpu
from jax.experimental.pallas import tpu_sc as plsc

VOCAB = 128
EMBED = 32
NC = 2
NS = 16
NW = NC * NS
BLK = 1024
DTILE = 8
UNIT_OUT = BLK * DTILE


@functools.cache
def _build(n_idx):
    n_units = (n_idx // BLK) * (EMBED // DTILE)
    u_per_w = n_units // NW

    mesh = plsc.VectorSubcoreMesh(core_axis_name="c", subcore_axis_name="s")

    @functools.partial(
        pl.kernel,
        mesh=mesh,
        out_type=jax.ShapeDtypeStruct((n_idx * EMBED,), jnp.float32),
        scratch_types=[
            pltpu.VMEM((VOCAB * EMBED,), jnp.float32),
            pltpu.VMEM((2 * BLK,), jnp.int32),
            pltpu.VMEM((2 * UNIT_OUT,), jnp.float32),
            pltpu.SemaphoreType.DMA,
            pltpu.SemaphoreType.DMA,
        ],
        compiler_params=pltpu.CompilerParams(
            use_tc_tiling_on_sc=False, needs_layout_passes=False),
    )
    def emb(idx_hbm, table_hbm, out_hbm, table_v, idx_v, out_v, idx_sem,
            out_sem):
        wid = lax.axis_index("s") * NC + lax.axis_index("c")
        base = wid * u_per_w
        pltpu.sync_copy(table_hbm, table_v)

        def idx_copy(u, buf):
            return pltpu.make_async_copy(
                idx_hbm.at[pl.ds((u >> 2) * BLK, BLK)],
                idx_v.at[pl.ds(buf * BLK, BLK)], idx_sem)

        def out_copy(u, buf):
            return pltpu.make_async_copy(
                out_v.at[pl.ds(buf * UNIT_OUT, UNIT_OUT)],
                out_hbm.at[pl.ds(u * UNIT_OUT, UNIT_OUT)], out_sem)

        def compute(u, p):
            d_base = (u & 3) * DTILE

            def group(g, c):
                iv = idx_v[pl.ds(p * BLK + g * 16, 16)] * EMBED
                boff = p * UNIT_OUT + (g >> 3) * 1024 + (g & 7) * 16
                for ds in range(DTILE):
                    vals = plsc.load_gather(table_v, [iv + (d_base + ds)])
                    out_v[pl.ds(boff + ds * 128, 16)] = vals
                return c

            lax.fori_loop(0, BLK // 16, group, 0)

        idx_copy(base, 0).start()
        idx_copy(base, 0).wait()
        idx_copy(base + 1, 1).start()
        compute(base, 0)
        out_copy(base, 0).start()
        idx_copy(base + 1, 1).wait()
        idx_copy(base + 2, 0).start()
        compute(base + 1, 1)
        out_copy(base + 1, 1).start()

        def step(j, c):
            u = base + j
            p = j & 1
            idx_copy(u, p).wait()
            idx_copy(u + 1, 1 - p).start()
            out_copy(u - 2, p).wait()
            compute(u, p)
            out_copy(u, p).start()
            return c

        lax.fori_loop(2, u_per_w - 1, step, 0)

        u = base + u_per_w - 1
        p = (u_per_w - 1) & 1
        idx_copy(u, p).wait()
        out_copy(u - 2, p).wait()
        compute(u, p)
        out_copy(u, p).start()
        out_copy(u - 1, 1 - p).wait()
        out_copy(u, p).wait()

    return emb


def kernel(inputs, table):
    NB, NT, NCH = inputs.shape
    idx = jnp.transpose(inputs, (1, 2, 0)).reshape(-1).astype(jnp.int32)
    flat = _build(idx.shape[0])(idx, table.reshape(-1))
    out = (
        flat.reshape(NT, NCH, EMBED // 8, NB // 128, 8, 128)
        .transpose(3, 5, 0, 1, 2, 4)
        .reshape(NB, NT, NCH, EMBED)
    )
    return out


# speedup vs baseline: 1.8743x; 1.8743x over previous
import functools

import jax
import jax.numpy as jnp
from jax import lax
from jax.experimental import pallas as pl
from jax.experimental.pallas import tpu as pltpu
from jax.experimental.pallas import tpu_sc as plsc

VOCAB = 128
EMBED = 32
NC = 2
NS = 16
NW = NC * NS
BLK = 1024
DTILE = 8
UNIT_OUT = BLK * DTILE


@functools.cache
def _build(n_idx):
    n_units = (n_idx // BLK) * (EMBED // DTILE)
    u_per_w = n_units // NW

    mesh = plsc.VectorSubcoreMesh(core_axis_name="c", subcore_axis_name="s")

    @functools.partial(
        pl.kernel,
        mesh=mesh,
        out_type=jax.ShapeDtypeStruct((n_idx * EMBED,), jnp.float32),
        scratch_types=[
            pltpu.VMEM((VOCAB * EMBED,), jnp.float32),
            pltpu.VMEM((2 * BLK,), jnp.int32),
            pltpu.VMEM((2 * UNIT_OUT,), jnp.float32),
            pltpu.SemaphoreType.DMA,
            pltpu.SemaphoreType.DMA,
        ],
        compiler_params=pltpu.CompilerParams(
            use_tc_tiling_on_sc=False, needs_layout_passes=False),
    )
    def emb(idx_hbm, table_hbm, out_hbm, table_v, idx_v, out_v, idx_sem,
            out_sem):
        wid = lax.axis_index("s") * NC + lax.axis_index("c")
        base = wid * u_per_w
        pltpu.sync_copy(table_hbm, table_v)

        def idx_copy(u, buf):
            return pltpu.make_async_copy(
                idx_hbm.at[pl.ds((u >> 2) * BLK, BLK)],
                idx_v.at[pl.ds(buf * BLK, BLK)], idx_sem)

        def out_copy(u, buf):
            return pltpu.make_async_copy(
                out_v.at[pl.ds(buf * UNIT_OUT, UNIT_OUT)],
                out_hbm.at[pl.ds(u * UNIT_OUT, UNIT_OUT)], out_sem)

        def compute(u, p):
            d_base = (u & 3) * DTILE

            @plsc.parallel_loop(0, BLK // 16, unroll=4)
            def group(g):
                iv = idx_v[pl.ds(p * BLK + g * 16, 16)] * EMBED + d_base
                boff = p * UNIT_OUT + (g >> 3) * 1024 + (g & 7) * 16
                vals = [
                    plsc.load_gather(table_v, [iv + ds])
                    for ds in range(DTILE)
                ]
                for ds in range(DTILE):
                    out_v[pl.ds(boff + ds * 128, 16)] = vals[ds]

        idx_copy(base, 0).start()
        idx_copy(base, 0).wait()
        idx_copy(base + 1, 1).start()
        compute(base, 0)
        out_copy(base, 0).start()
        idx_copy(base + 1, 1).wait()
        idx_copy(base + 2, 0).start()
        compute(base + 1, 1)
        out_copy(base + 1, 1).start()

        def step(j, c):
            u = base + j
            p = j & 1
            idx_copy(u, p).wait()
            idx_copy(u + 1, 1 - p).start()
            out_copy(u - 2, p).wait()
            compute(u, p)
            out_copy(u, p).start()
            return c

        lax.fori_loop(2, u_per_w - 1, step, 0)

        u = base + u_per_w - 1
        p = (u_per_w - 1) & 1
        idx_copy(u, p).wait()
        out_copy(u - 2, p).wait()
        compute(u, p)
        out_copy(u, p).start()
        out_copy(u - 1, 1 - p).wait()
        out_copy(u, p).wait()

    return emb


def kernel(inputs, table):
    NB, NT, NCH = inputs.shape
    idx = jnp.transpose(inputs, (1, 2, 0)).reshape(-1).astype(jnp.int32)
    flat = _build(idx.shape[0])(idx, table.reshape(-1))
    out = (
        flat.reshape(NT, NCH, EMBED // 8, NB // 128, 8, 128)
        .transpose(3, 5, 0, 1, 2, 4)
        .reshape(NB, NT, NCH, EMBED)
    )
    return out


# speedup vs baseline: 6.6738x; 3.5607x over previous
import functools

import jax
import jax.numpy as jnp
from jax import lax
from jax.experimental import pallas as pl
from jax.experimental.pallas import tpu as pltpu
from jax.experimental.pallas import tpu_sc as plsc

VOCAB = 128
EMBED = 32
NC = 2
NS = 16
NW = NC * NS
BLK = 1024
DTILE = 8
UNIT_OUT = BLK * DTILE


@functools.cache
def _build(n_idx):
    n_units = (n_idx // BLK) * (EMBED // DTILE)
    u_per_w = n_units // NW

    mesh = plsc.VectorSubcoreMesh(core_axis_name="c", subcore_axis_name="s")

    @functools.partial(
        pl.kernel,
        mesh=mesh,
        out_type=jax.ShapeDtypeStruct((n_idx * EMBED,), jnp.float32),
        scratch_types=[
            pltpu.VMEM((VOCAB * EMBED,), jnp.float32),
            pltpu.VMEM((2 * BLK,), jnp.int32),
            pltpu.VMEM((2 * UNIT_OUT,), jnp.float32),
            pltpu.SemaphoreType.DMA,
            pltpu.SemaphoreType.DMA,
        ],
        compiler_params=pltpu.CompilerParams(
            use_tc_tiling_on_sc=False, needs_layout_passes=False),
    )
    def emb(idx_hbm, table_hbm, out_hbm, table_v, idx_v, out_v, idx_sem,
            out_sem):
        wid = lax.axis_index("s") * NC + lax.axis_index("c")
        base = wid * u_per_w
        pltpu.sync_copy(table_hbm, table_v)

        def idx_copy(u, buf):
            return pltpu.make_async_copy(
                idx_hbm.at[pl.ds((u >> 2) * BLK, BLK)],
                idx_v.at[pl.ds(buf * BLK, BLK)], idx_sem)

        def out_copy(u, buf):
            return pltpu.make_async_copy(
                out_v.at[pl.ds(buf * UNIT_OUT, UNIT_OUT)],
                out_hbm.at[pl.ds(u * UNIT_OUT, UNIT_OUT)], out_sem)

        def compute(u, p):
            d_base = (u & 3) * DTILE

            @plsc.parallel_loop(0, BLK // 16, unroll=4)
            def group(g):
                iv = idx_v[pl.ds(p * BLK + g * 16, 16)] + d_base * VOCAB
                boff = p * UNIT_OUT + (g >> 3) * 1024 + (g & 7) * 16
                vals = [
                    plsc.load_gather(table_v, [iv + ds * VOCAB])
                    for ds in range(DTILE)
                ]
                for ds in range(DTILE):
                    out_v[pl.ds(boff + ds * 128, 16)] = vals[ds]

        idx_copy(base, 0).start()
        idx_copy(base, 0).wait()
        idx_copy(base + 1, 1).start()
        compute(base, 0)
        out_copy(base, 0).start()
        idx_copy(base + 1, 1).wait()
        idx_copy(base + 2, 0).start()
        compute(base + 1, 1)
        out_copy(base + 1, 1).start()

        def step(j, c):
            u = base + j
            p = j & 1
            idx_copy(u, p).wait()
            idx_copy(u + 1, 1 - p).start()
            out_copy(u - 2, p).wait()
            compute(u, p)
            out_copy(u, p).start()
            return c

        lax.fori_loop(2, u_per_w - 1, step, 0)

        u = base + u_per_w - 1
        p = (u_per_w - 1) & 1
        idx_copy(u, p).wait()
        out_copy(u - 2, p).wait()
        compute(u, p)
        out_copy(u, p).start()
        out_copy(u - 1, 1 - p).wait()
        out_copy(u, p).wait()

    return emb


def kernel(inputs, table):
    NB, NT, NCH = inputs.shape
    idx = jnp.transpose(inputs, (1, 2, 0)).reshape(-1).astype(jnp.int32)
    flat = _build(idx.shape[0])(idx, jnp.transpose(table).reshape(-1))
    out = (
        flat.reshape(NT, NCH, EMBED // 8, NB // 128, 8, 128)
        .transpose(3, 5, 0, 1, 2, 4)
        .reshape(NB, NT, NCH, EMBED)
    )
    return out
